# baseline (speedup 1.0000x reference)
import functools

import jax
import jax.numpy as jnp
from jax import lax
from jax.experimental import pallas as pl
from jax.experimental.pallas import tpu as pltpu
from jax.experimental.pallas import tpu_sc as plsc

B = 16384
F = 26
V = 100000
D = 32
NUM_DIM = 13
FP = 28
G = FP // 4
ROWS2 = B * FP
QROWS = F * V // 4

_NW = 32

_mesh = plsc.VectorSubcoreMesh(core_axis_name="c", subcore_axis_name="s")


WQ = 25088
WC = 3584
VR = 4 * WQ


def _tc_repack_body(x0, x1, x2, x3, out_ref):
    for j, xj in enumerate((x0, x1, x2, x3)):
        out_ref[0, :, j * 32:(j + 1) * 32] = xj[0].T


def _tc_repack(tabT):
    def in_spec(j):
        return pl.BlockSpec((1, D, WC), lambda f, c, j=j: (f, 0, j * (WQ // WC) + c))

    return pl.pallas_call(
        _tc_repack_body,
        grid=(F, WQ // WC),
        in_specs=[in_spec(0), in_spec(1), in_spec(2), in_spec(3)],
        out_specs=pl.BlockSpec((1, WC, 128), lambda f, c: (f, c, 0)),
        out_shape=jax.ShapeDtypeStruct((F, WQ, 128), jnp.float32),
    )(tabT, tabT, tabT, tabT)


_CH = 1024
_PER_W = ROWS2 // _NW
_NCH = _PER_W // _CH


@functools.partial(
    pl.kernel,
    mesh=_mesh,
    out_type=jax.ShapeDtypeStruct((ROWS2, D), jnp.float32),
    scratch_types=[
        pltpu.VMEM((_CH,), jnp.int32),
        pltpu.VMEM((_CH,), jnp.int32),
        pltpu.VMEM((_CH, D), jnp.float32),
        pltpu.VMEM((_CH, D), jnp.float32),
        pltpu.SemaphoreType.DMA,
        pltpu.SemaphoreType.DMA,
        pltpu.SemaphoreType.DMA,
        pltpu.SemaphoreType.DMA,
    ],
    compiler_params=pltpu.CompilerParams(use_tc_tiling_on_sc=False,
                                         needs_layout_passes=False),
)
def _sc_gather(idx_hbm, tp_hbm, out_hbm, idx0, idx1, rows0, rows1,
               si0, si1, so0, so1):
    wid = lax.axis_index("s") * 2 + lax.axis_index("c")
    base = pl.multiple_of(wid * _PER_W, 1024)
    idxs = (idx0, idx1)
    rows = (rows0, rows1)
    sis = (si0, si1)
    sos = (so0, so1)

    def start_in(c, b):
        off = pl.multiple_of(base + c * _CH, 1024)
        pltpu.async_copy(idx_hbm.at[pl.ds(off, _CH)], idxs[b], sis[b])

    start_in(0, 0)
    start_in(1, 1)

    def chunk_pair(i2, carry):
        for b in range(2):
            c = i2 * 2 + b
            pltpu.make_async_copy(idx_hbm.at[pl.ds(0, _CH)], idxs[b],
                                  sis[b]).wait()

            @pl.when(c >= 2)
            def _():
                pltpu.make_async_copy(rows[b],
                                      out_hbm.at[pl.ds(0, _CH)],
                                      sos[b]).wait()

            pltpu.async_copy(tp_hbm.at[idxs[b]], rows[b], sos[b]).wait()
            off = pl.multiple_of(base + c * _CH, 1024)
            pltpu.async_copy(rows[b], out_hbm.at[pl.ds(off, _CH)], sos[b])

            @pl.when(c + 2 < _NCH)
            def _():
                start_in(c + 2, b)

        return carry

    lax.fori_loop(0, _NCH // 2, chunk_pair, 0)
    for b in range(2):
        pltpu.make_async_copy(rows[b], out_hbm.at[pl.ds(0, _CH)], sos[b]).wait()


def _mlp_body(emb_ref, num_ref, w1g_ref, w1n_ref, b1_ref, w2_ref, b2_ref, out_ref):
    h = jnp.dot(num_ref[...], w1n_ref[...], preferred_element_type=jnp.float32)
    for g in range(G):
        h = h + jnp.dot(emb_ref[g], w1g_ref[g], preferred_element_type=jnp.float32)
    h = jnp.maximum(h + b1_ref[...], 0.0)
    o = jnp.dot(h, w2_ref[...], preferred_element_type=jnp.float32)
    out_ref[...] = jnp.maximum(o + b2_ref[...], 0.0)


_BB = 2048


def _mlp(emb3, num, w1g, w1n, b1, w2, b2):
    return pl.pallas_call(
        _mlp_body,
        grid=(B // _BB,),
        in_specs=[
            pl.BlockSpec((G, _BB, 128), lambda i: (0, i, 0)),
            pl.BlockSpec((_BB, NUM_DIM), lambda i: (i, 0)),
            pl.BlockSpec((G, 128, 64), lambda i: (0, 0, 0)),
            pl.BlockSpec((NUM_DIM, 64), lambda i: (0, 0)),
            pl.BlockSpec((1, 64), lambda i: (0, 0)),
            pl.BlockSpec((64, 32), lambda i: (0, 0)),
            pl.BlockSpec((1, 32), lambda i: (0, 0)),
        ],
        out_specs=pl.BlockSpec((_BB, 32), lambda i: (i, 0)),
        out_shape=jax.ShapeDtypeStruct((B, 32), jnp.float32),
    )(emb3, num, w1g, w1n, b1, w2, b2)


def kernel(cate_inputs, num_inputs, tables, W1, b1, W2, b2):
    tabT = jnp.transpose(tables, (0, 2, 1))
    tp = _tc_repack(tabT)
    tp_rows = tp.reshape(F * VR, D)

    f_ar = jnp.arange(FP, dtype=jnp.int32)
    bases = jnp.where(f_ar < F, f_ar * VR, 0)
    cate_p = jnp.pad(cate_inputs.astype(jnp.int32), ((0, 0), (0, FP - F)))
    cate_r = (cate_p % WQ) * 4 + cate_p // WQ
    idx = (cate_r + bases[None, :]).reshape(B, G, 4)
    idx = jnp.transpose(idx, (1, 0, 2)).reshape(ROWS2)

    emb = _sc_gather(idx, tp_rows)
    emb3 = emb.reshape(G, B, 128)

    w1e = W1[:F * D]
    w1g = jnp.concatenate([w1e, jnp.zeros((FP * D - F * D, 64), jnp.float32)]).reshape(G, 128, 64)
    return _mlp(emb3, num_inputs, w1g, W1[F * D:], b1.reshape(1, 64),
                W2, b2.reshape(1, 32))
